# Initial kernel scaffold; baseline (speedup 1.0000x reference)
import functools

import jax
import jax.numpy as jnp
from jax import lax
from jax.experimental import pallas as pl
from jax.experimental.pallas import tpu as pltpu
from jax.experimental.pallas import tpu_sc as plsc

N_NODES = 10000
N_EDGES = 320000
D = 128

NC, NS, L = 2, 16, 16
NW = NC * NS
K = 128
BPT = 79
EP = NW * BPT * K
N_PAD = 10240
NR = N_PAD // NS


def _sc_mesh():
    return plsc.VectorSubcoreMesh(core_axis_name="c", subcore_axis_name="s")


@functools.partial(
    pl.kernel,
    out_type=jax.ShapeDtypeStruct((NC, N_PAD), jnp.float32),
    mesh=_sc_mesh(),
    scratch_types=[
        pltpu.VMEM((K,), jnp.int32),
        pltpu.VMEM((K,), jnp.float32),
        pltpu.VMEM((NR,), jnp.float32),
        pltpu.VMEM_SHARED((N_PAD,), jnp.float32),
    ],
)
def _sc_degree(dst_hbm, w_hbm, out_hbm, didx_v, w_v, z_v, deg_sh):
    c = lax.axis_index("c")
    s = lax.axis_index("s")
    wid = s * NC + c

    zero = jnp.zeros((L,), jnp.float32)
    for j in range(NR // L):
        z_v[pl.ds(j * L, L)] = zero
    pltpu.sync_copy(z_v, deg_sh.at[pl.ds(s * NR, NR)])
    plsc.subcore_barrier()

    def body(b, carry):
        base = (wid * BPT + b) * K
        pltpu.sync_copy(dst_hbm.at[pl.ds(base, K)], didx_v)
        pltpu.sync_copy(w_hbm.at[pl.ds(base, K)], w_v)
        pltpu.sync_copy(w_v, deg_sh.at[didx_v], add=True)
        return carry

    lax.fori_loop(0, BPT, body, 0)
    plsc.subcore_barrier()
    pltpu.sync_copy(deg_sh.at[pl.ds(s * NR, NR)], out_hbm.at[c, pl.ds(s * NR, NR)])


@functools.partial(
    pl.kernel,
    out_type=jax.ShapeDtypeStruct((NC, N_PAD, D), jnp.float32),
    mesh=_sc_mesh(),
    scratch_types=[
        pltpu.VMEM((K,), jnp.int32),
        pltpu.VMEM((K,), jnp.int32),
        pltpu.VMEM((K,), jnp.float32),
        pltpu.VMEM((K, D), jnp.float32),
        pltpu.VMEM_SHARED((N_PAD, D), jnp.float32),
        pltpu.SemaphoreType.DMA,
    ],
)
def _sc_edge_pass(y_hbm, src_hbm, dst_hbm, w_hbm, out_hbm,
                  sidx_v, didx_v, w_v, rows_v, acc_sh, sem):
    c = lax.axis_index("c")
    s = lax.axis_index("s")
    wid = s * NC + c

    zero = jnp.zeros((L,), jnp.float32)

    def zero_rows(r, carry):
        for j in range(D // L):
            rows_v[r, pl.ds(j * L, L)] = zero
        return carry

    lax.fori_loop(0, K, zero_rows, 0)
    for t in range(NR // K):
        pltpu.sync_copy(rows_v, acc_sh.at[pl.ds(s * NR + t * K, K)])
    plsc.subcore_barrier()

    def body(b, carry):
        base = (wid * BPT + b) * K
        pltpu.sync_copy(src_hbm.at[pl.ds(base, K)], sidx_v)
        pltpu.sync_copy(dst_hbm.at[pl.ds(base, K)], didx_v)
        pltpu.sync_copy(w_hbm.at[pl.ds(base, K)], w_v)
        pltpu.async_copy(y_hbm.at[sidx_v], rows_v, sem).wait()

        def scale(k, inner):
            wk = w_v[k]
            for j in range(D // L):
                rows_v[k, pl.ds(j * L, L)] = rows_v[k, pl.ds(j * L, L)] * wk
            return inner

        lax.fori_loop(0, K, scale, 0)
        pltpu.sync_copy(rows_v, acc_sh.at[didx_v], add=True)
        return carry

    lax.fori_loop(0, BPT, body, 0)
    plsc.subcore_barrier()
    for t in range(NR // K):
        pltpu.sync_copy(acc_sh.at[pl.ds(s * NR + t * K, K)],
                        out_hbm.at[c, pl.ds(s * NR + t * K, K)])


def _tc_prescale_body(degp_ref, x_ref, dis_ref, y_ref):
    deg = degp_ref[0] + degp_ref[1]
    pos = deg > 0.0
    dis = jnp.where(pos, lax.rsqrt(jnp.where(pos, deg, 1.0)), 0.0)
    dis_ref[...] = dis
    y_ref[...] = dis * x_ref[...]


def _tc_rescale_body(dis_ref, p_ref, y_ref, *, square):
    dis = dis_ref[...]
    scale = dis * dis if square else dis
    y_ref[...] = scale * (p_ref[0] + p_ref[1])


_tc_prescale = pl.pallas_call(
    _tc_prescale_body,
    out_shape=(
        jax.ShapeDtypeStruct((N_PAD, 1), jnp.float32),
        jax.ShapeDtypeStruct((N_PAD, D), jnp.float32),
    ),
)

_tc_rescale_mid = pl.pallas_call(
    functools.partial(_tc_rescale_body, square=True),
    out_shape=jax.ShapeDtypeStruct((N_PAD, D), jnp.float32),
)

_tc_rescale_final = pl.pallas_call(
    functools.partial(_tc_rescale_body, square=False),
    out_shape=jax.ShapeDtypeStruct((N_PAD, D), jnp.float32),
)


@jax.jit
def kernel(x, edge_index, edge_weight):
    src = edge_index[0].astype(jnp.int32)
    dst = edge_index[1].astype(jnp.int32)
    srcp = jnp.pad(src, (0, EP - N_EDGES))
    dstp = jnp.pad(dst, (0, EP - N_EDGES))
    wp = jnp.pad(edge_weight, (0, EP - N_EDGES))
    xp = jnp.pad(x, ((0, N_PAD - N_NODES), (0, 0)))

    degp = _sc_degree(dstp, wp)
    dis, y1 = _tc_prescale(degp[:, :, None], xp)
    p = _sc_edge_pass(y1, srcp, dstp, wp)
    y2 = _tc_rescale_mid(dis, p)
    q = _sc_edge_pass(y2, srcp, dstp, wp)
    h = _tc_rescale_final(dis, q)
    return h[:N_NODES]


# speedup vs baseline: 7.5190x; 7.5190x over previous
import functools

import jax
import jax.numpy as jnp
from jax import lax
from jax.experimental import pallas as pl
from jax.experimental.pallas import tpu as pltpu
from jax.experimental.pallas import tpu_sc as plsc

N_NODES = 10000
N_EDGES = 320000
D = 128

NC, NS, L = 2, 16, 16
NW = NC * NS
K = 128
BPT = 79
EP = NW * BPT * K
N_PAD = 10240
NR = N_PAD // NS


def _sc_mesh():
    return plsc.VectorSubcoreMesh(core_axis_name="c", subcore_axis_name="s")


@functools.partial(
    pl.kernel,
    out_type=jax.ShapeDtypeStruct((NC, N_PAD), jnp.float32),
    mesh=_sc_mesh(),
    scratch_types=[
        pltpu.VMEM((K,), jnp.int32),
        pltpu.VMEM((K,), jnp.float32),
        pltpu.VMEM((NR,), jnp.float32),
        pltpu.VMEM_SHARED((N_PAD,), jnp.float32),
    ],
)
def _sc_degree(dst_hbm, w_hbm, out_hbm, didx_v, w_v, z_v, deg_sh):
    c = lax.axis_index("c")
    s = lax.axis_index("s")
    wid = s * NC + c

    zero = jnp.zeros((L,), jnp.float32)
    for j in range(NR // L):
        z_v[pl.ds(j * L, L)] = zero
    pltpu.sync_copy(z_v, deg_sh.at[pl.ds(s * NR, NR)])
    plsc.subcore_barrier()

    def body(b, carry):
        base = (wid * BPT + b) * K
        pltpu.sync_copy(dst_hbm.at[pl.ds(base, K)], didx_v)
        pltpu.sync_copy(w_hbm.at[pl.ds(base, K)], w_v)
        pltpu.sync_copy(w_v, deg_sh.at[didx_v], add=True)
        return carry

    lax.fori_loop(0, BPT, body, 0)
    plsc.subcore_barrier()
    pltpu.sync_copy(deg_sh.at[pl.ds(s * NR, NR)], out_hbm.at[c, pl.ds(s * NR, NR)])


@functools.partial(
    pl.kernel,
    out_type=jax.ShapeDtypeStruct((NC, N_PAD, D), jnp.float32),
    mesh=_sc_mesh(),
    scratch_types=[
        pltpu.VMEM((K,), jnp.int32),
        pltpu.VMEM((K,), jnp.int32),
        pltpu.VMEM((K,), jnp.float32),
        pltpu.VMEM((K, D), jnp.float32),
        pltpu.VMEM_SHARED((N_PAD, D), jnp.float32),
        pltpu.SemaphoreType.DMA,
    ],
)
def _sc_edge_pass(y_hbm, src_hbm, dst_hbm, w_hbm, out_hbm,
                  sidx_v, didx_v, w_v, rows_v, acc_sh, sem):
    c = lax.axis_index("c")
    s = lax.axis_index("s")
    wid = s * NC + c

    zero = jnp.zeros((L,), jnp.float32)

    def zero_rows(r, carry):
        for j in range(D // L):
            rows_v[r, pl.ds(j * L, L)] = zero
        return carry

    lax.fori_loop(0, K, zero_rows, 0)
    for t in range(NR // K):
        pltpu.sync_copy(rows_v, acc_sh.at[pl.ds(s * NR + t * K, K)])
    plsc.subcore_barrier()

    def body(b, carry):
        base = (wid * BPT + b) * K
        pltpu.sync_copy(src_hbm.at[pl.ds(base, K)], sidx_v)
        pltpu.sync_copy(dst_hbm.at[pl.ds(base, K)], didx_v)
        pltpu.sync_copy(w_hbm.at[pl.ds(base, K)], w_v)
        pltpu.async_copy(y_hbm.at[sidx_v], rows_v, sem).wait()

        def scale(g, inner):
            wg = w_v[pl.ds(g * L, L)]
            for i in range(L):
                k = g * L + i
                wk = wg[i]
                for j in range(D // L):
                    rows_v[k, pl.ds(j * L, L)] = rows_v[k, pl.ds(j * L, L)] * wk
            return inner

        lax.fori_loop(0, K // L, scale, 0)
        pltpu.sync_copy(rows_v, acc_sh.at[didx_v], add=True)
        return carry

    lax.fori_loop(0, BPT, body, 0)
    plsc.subcore_barrier()
    for t in range(NR // K):
        pltpu.sync_copy(acc_sh.at[pl.ds(s * NR + t * K, K)],
                        out_hbm.at[c, pl.ds(s * NR + t * K, K)])


def _tc_prescale_body(degp_ref, x_ref, dis_ref, y_ref):
    deg = degp_ref[0] + degp_ref[1]
    pos = deg > 0.0
    dis = jnp.where(pos, lax.rsqrt(jnp.where(pos, deg, 1.0)), 0.0)
    dis_ref[...] = dis
    y_ref[...] = dis * x_ref[...]


def _tc_rescale_body(dis_ref, p_ref, y_ref, *, square):
    dis = dis_ref[...]
    scale = dis * dis if square else dis
    y_ref[...] = scale * (p_ref[0] + p_ref[1])


_tc_prescale = pl.pallas_call(
    _tc_prescale_body,
    out_shape=(
        jax.ShapeDtypeStruct((N_PAD, 1), jnp.float32),
        jax.ShapeDtypeStruct((N_PAD, D), jnp.float32),
    ),
)

_tc_rescale_mid = pl.pallas_call(
    functools.partial(_tc_rescale_body, square=True),
    out_shape=jax.ShapeDtypeStruct((N_PAD, D), jnp.float32),
)

_tc_rescale_final = pl.pallas_call(
    functools.partial(_tc_rescale_body, square=False),
    out_shape=jax.ShapeDtypeStruct((N_PAD, D), jnp.float32),
)


@jax.jit
def kernel(x, edge_index, edge_weight):
    src = edge_index[0].astype(jnp.int32)
    dst = edge_index[1].astype(jnp.int32)
    srcp = jnp.pad(src, (0, EP - N_EDGES))
    dstp = jnp.pad(dst, (0, EP - N_EDGES))
    wp = jnp.pad(edge_weight, (0, EP - N_EDGES))
    xp = jnp.pad(x, ((0, N_PAD - N_NODES), (0, 0)))

    degp = _sc_degree(dstp, wp)
    dis, y1 = _tc_prescale(degp[:, :, None], xp)
    p = _sc_edge_pass(y1, srcp, dstp, wp)
    y2 = _tc_rescale_mid(dis, p)
    q = _sc_edge_pass(y2, srcp, dstp, wp)
    h = _tc_rescale_final(dis, q)
    return h[:N_NODES]


# speedup vs baseline: 7.5784x; 1.0079x over previous
import functools

import jax
import jax.numpy as jnp
from jax import lax
from jax.experimental import pallas as pl
from jax.experimental.pallas import tpu as pltpu
from jax.experimental.pallas import tpu_sc as plsc

N_NODES = 10000
N_EDGES = 320000
D = 128
DH = 64

NC, NS, L = 2, 16, 16
NW = NC * NS
K = 128
BPT = 80
EP = NW * BPT * K
N_PAD = 10240
NR = N_PAD // NS


def _sc_mesh():
    return plsc.VectorSubcoreMesh(core_axis_name="c", subcore_axis_name="s")


@functools.partial(
    pl.kernel,
    out_type=jax.ShapeDtypeStruct((NC, N_PAD), jnp.float32),
    mesh=_sc_mesh(),
    scratch_types=[
        pltpu.VMEM((BPT, K), jnp.int32),
        pltpu.VMEM((BPT, K), jnp.float32),
        pltpu.VMEM((NR,), jnp.float32),
        pltpu.VMEM_SHARED((N_PAD,), jnp.float32),
    ],
)
def _sc_degree(dst_hbm, w_hbm, out_hbm, didx_all, w_all, z_v, deg_sh):
    c = lax.axis_index("c")
    s = lax.axis_index("s")
    wid = s * NC + c

    pltpu.sync_copy(dst_hbm.at[pl.ds(wid * BPT, BPT)], didx_all)
    pltpu.sync_copy(w_hbm.at[pl.ds(wid * BPT, BPT)], w_all)

    zero = jnp.zeros((L,), jnp.float32)
    for j in range(NR // L):
        z_v[pl.ds(j * L, L)] = zero
    pltpu.sync_copy(z_v, deg_sh.at[pl.ds(s * NR, NR)])
    plsc.subcore_barrier()

    def body(b, carry):
        pltpu.sync_copy(w_all.at[b], deg_sh.at[didx_all.at[b]], add=True)
        return carry

    lax.fori_loop(0, BPT, body, 0)
    plsc.subcore_barrier()
    pltpu.sync_copy(deg_sh.at[pl.ds(s * NR, NR)], out_hbm.at[c, pl.ds(s * NR, NR)])


@functools.partial(
    pl.kernel,
    out_type=jax.ShapeDtypeStruct((NC, N_PAD, DH), jnp.float32),
    mesh=_sc_mesh(),
    scratch_types=[
        pltpu.VMEM((BPT, K), jnp.int32),
        pltpu.VMEM((BPT, K), jnp.int32),
        pltpu.VMEM((BPT, K), jnp.float32),
        pltpu.VMEM((K, DH), jnp.float32),
        pltpu.VMEM((K, DH), jnp.float32),
        pltpu.VMEM((K, DH), jnp.float32),
        pltpu.VMEM_SHARED((N_PAD, DH), jnp.float32),
        pltpu.SemaphoreType.DMA,
        pltpu.SemaphoreType.DMA,
        pltpu.SemaphoreType.DMA,
        pltpu.SemaphoreType.DMA,
        pltpu.SemaphoreType.DMA,
        pltpu.SemaphoreType.DMA,
    ],
    compiler_params=pltpu.CompilerParams(use_tc_tiling_on_sc=False),
)
def _sc_edge_pass(y_hbm, src_hbm, dst_hbm, w_hbm, out_hbm,
                  sidx_all, didx_all, w_all, rows0, rows1, rows2, acc_sh,
                  g0, g1, g2, s0, s1, s2):
    c = lax.axis_index("c")
    s = lax.axis_index("s")
    wid = s * NC + c
    rows = (rows0, rows1, rows2)
    gsem = (g0, g1, g2)
    ssem = (s0, s1, s2)

    pltpu.sync_copy(src_hbm.at[pl.ds(wid * BPT, BPT)], sidx_all)
    pltpu.sync_copy(dst_hbm.at[pl.ds(wid * BPT, BPT)], didx_all)
    pltpu.sync_copy(w_hbm.at[pl.ds(wid * BPT, BPT)], w_all)

    zero = jnp.zeros((L,), jnp.float32)

    def zero_rows(r, carry):
        for j in range(DH // L):
            rows0[r, pl.ds(j * L, L)] = zero
        return carry

    lax.fori_loop(0, K, zero_rows, 0)
    for t in range(NR // K):
        pltpu.sync_copy(rows0, acc_sh.at[pl.ds(s * NR + t * K, K)])
    plsc.subcore_barrier()

    def gather(b, p):
        pltpu.async_copy(y_hbm.at[sidx_all.at[b]], rows[p], gsem[p])

    def gather_wait(b, p):
        pltpu.make_async_copy(y_hbm.at[sidx_all.at[b]], rows[p], gsem[p]).wait()

    def scatter(b, p):
        pltpu.async_copy(rows[p], acc_sh.at[didx_all.at[b]], ssem[p], add=True)

    def scatter_wait(b, p):
        pltpu.make_async_copy(rows[p], acc_sh.at[didx_all.at[b]], ssem[p]).wait()

    def scale(b, p):
        def g_body(g, carry):
            wg = w_all[b, pl.ds(g * L, L)]
            for i in range(L):
                wk = wg[i]
                k = g * L + i
                for j in range(DH // L):
                    rows[p][k, pl.ds(j * L, L)] = rows[p][k, pl.ds(j * L, L)] * wk
            return carry

        lax.fori_loop(0, K // L, g_body, 0)

    gather(0, 0)
    gather(1, 1)

    def body(i, carry):
        for u in range(3):
            b = 3 * i + u
            p = u
            pn = (u + 2) % 3
            gather_wait(b, p)
            scale(b, p)
            scatter(b, p)

            @pl.when(b >= 1)
            def _():
                scatter_wait(b - 1, pn)

            @pl.when(b + 2 < BPT)
            def _():
                gather(b + 2, pn)
        return carry

    n_main = BPT // 3
    lax.fori_loop(0, n_main, body, 0)

    for b in range(3 * n_main, BPT):
        p = b % 3
        gather_wait(b, p)
        scale(b, p)
        scatter(b, p)
        scatter_wait(b - 1, (b - 1) % 3)
    scatter_wait(BPT - 1, (BPT - 1) % 3)
    plsc.subcore_barrier()
    for t in range(NR // K):
        pltpu.sync_copy(acc_sh.at[pl.ds(s * NR + t * K, K)],
                        out_hbm.at[c, pl.ds(s * NR + t * K, K)])


def _tc_prescale_body(degp_ref, x_ref, dis_ref, ya_ref, yb_ref):
    deg = degp_ref[0] + degp_ref[1]
    pos = deg > 0.0
    dis = jnp.where(pos, lax.rsqrt(jnp.where(pos, deg, 1.0)), 0.0)
    dis_ref[...] = dis
    ya_ref[...] = dis * x_ref[:, :DH]
    yb_ref[...] = dis * x_ref[:, DH:]


def _tc_mid_body(dis_ref, pa_ref, pb_ref, ya_ref, yb_ref):
    d2 = dis_ref[...] * dis_ref[...]
    ya_ref[...] = d2 * (pa_ref[0] + pa_ref[1])
    yb_ref[...] = d2 * (pb_ref[0] + pb_ref[1])


def _tc_final_body(dis_ref, qa_ref, qb_ref, h_ref):
    dis = dis_ref[...]
    h_ref[:, :DH] = dis * (qa_ref[0] + qa_ref[1])
    h_ref[:, DH:] = dis * (qb_ref[0] + qb_ref[1])


_tc_prescale = pl.pallas_call(
    _tc_prescale_body,
    out_shape=(
        jax.ShapeDtypeStruct((N_PAD, 1), jnp.float32),
        jax.ShapeDtypeStruct((N_PAD, DH), jnp.float32),
        jax.ShapeDtypeStruct((N_PAD, DH), jnp.float32),
    ),
)

_tc_mid = pl.pallas_call(
    _tc_mid_body,
    out_shape=(
        jax.ShapeDtypeStruct((N_PAD, DH), jnp.float32),
        jax.ShapeDtypeStruct((N_PAD, DH), jnp.float32),
    ),
)

_tc_final = pl.pallas_call(
    _tc_final_body,
    out_shape=jax.ShapeDtypeStruct((N_PAD, D), jnp.float32),
)


@jax.jit
def kernel(x, edge_index, edge_weight):
    src = edge_index[0].astype(jnp.int32)
    dst = edge_index[1].astype(jnp.int32)
    srcp = jnp.pad(src, (0, EP - N_EDGES)).reshape(NW * BPT, K)
    dstp = jnp.pad(dst, (0, EP - N_EDGES)).reshape(NW * BPT, K)
    wp = jnp.pad(edge_weight, (0, EP - N_EDGES)).reshape(NW * BPT, K)
    xp = jnp.pad(x, ((0, N_PAD - N_NODES), (0, 0)))

    degp = _sc_degree(dstp, wp)
    dis, ya, yb = _tc_prescale(degp[:, :, None], xp)
    pa = _sc_edge_pass(ya, srcp, dstp, wp)
    pb = _sc_edge_pass(yb, srcp, dstp, wp)
    y2a, y2b = _tc_mid(dis, pa, pb)
    qa = _sc_edge_pass(y2a, srcp, dstp, wp)
    qb = _sc_edge_pass(y2b, srcp, dstp, wp)
    h = _tc_final(dis, qa, qb)
    return h[:N_NODES]


# speedup vs baseline: 8.7428x; 1.1536x over previous
import functools

import jax
import jax.numpy as jnp
from jax import lax
from jax.experimental import pallas as pl
from jax.experimental.pallas import tpu as pltpu
from jax.experimental.pallas import tpu_sc as plsc

N_NODES = 10000
N_EDGES = 320000
D = 128
DH = 64

NC, NS, L = 2, 16, 16
NW = NC * NS
K = 128
BPT = 80
EP = NW * BPT * K
N_PAD = 10240
NR = N_PAD // NS


def _sc_mesh():
    return plsc.VectorSubcoreMesh(core_axis_name="c", subcore_axis_name="s")


@functools.partial(
    pl.kernel,
    out_type=jax.ShapeDtypeStruct((NC, N_PAD), jnp.float32),
    mesh=_sc_mesh(),
    scratch_types=[
        pltpu.VMEM((BPT, K), jnp.int32),
        pltpu.VMEM((BPT, K), jnp.float32),
        pltpu.VMEM((NR,), jnp.float32),
        pltpu.VMEM_SHARED((N_PAD,), jnp.float32),
    ],
)
def _sc_degree(dst_hbm, w_hbm, out_hbm, didx_all, w_all, z_v, deg_sh):
    c = lax.axis_index("c")
    s = lax.axis_index("s")
    wid = s * NC + c

    pltpu.sync_copy(dst_hbm.at[pl.ds(wid * BPT, BPT)], didx_all)
    pltpu.sync_copy(w_hbm.at[pl.ds(wid * BPT, BPT)], w_all)

    zero = jnp.zeros((L,), jnp.float32)
    for j in range(NR // L):
        z_v[pl.ds(j * L, L)] = zero
    pltpu.sync_copy(z_v, deg_sh.at[pl.ds(s * NR, NR)])
    plsc.subcore_barrier()

    def body(b, carry):
        pltpu.sync_copy(w_all.at[b], deg_sh.at[didx_all.at[b]], add=True)
        return carry

    lax.fori_loop(0, BPT, body, 0)
    plsc.subcore_barrier()
    pltpu.sync_copy(deg_sh.at[pl.ds(s * NR, NR)], out_hbm.at[c, pl.ds(s * NR, NR)])


@functools.partial(
    pl.kernel,
    out_type=jax.ShapeDtypeStruct((NC, N_PAD, DH), jnp.float32),
    mesh=_sc_mesh(),
    scratch_types=[
        pltpu.VMEM((BPT, K), jnp.int32),
        pltpu.VMEM((BPT, K), jnp.int32),
        pltpu.VMEM((BPT, K), jnp.float32),
        pltpu.VMEM((K, DH), jnp.float32),
        pltpu.VMEM((K, DH), jnp.float32),
        pltpu.VMEM_SHARED((N_PAD, DH), jnp.float32),
        pltpu.VMEM_SHARED((N_PAD, DH), jnp.float32),
        pltpu.SemaphoreType.DMA,
        pltpu.SemaphoreType.DMA,
        pltpu.SemaphoreType.DMA,
        pltpu.SemaphoreType.DMA,
    ],
    compiler_params=pltpu.CompilerParams(use_tc_tiling_on_sc=False),
)
def _sc_edge_pass(y_hbm, src_hbm, dst_hbm, w_hbm, out_hbm,
                  sidx_all, didx_all, w_all, rows0, rows1, y_sh, acc_sh,
                  g0, g1, s0, s1):
    c = lax.axis_index("c")
    s = lax.axis_index("s")
    wid = s * NC + c
    rows = (rows0, rows1)
    gsem = (g0, g1)
    ssem = (s0, s1)

    pltpu.sync_copy(src_hbm.at[pl.ds(wid * BPT, BPT)], sidx_all)
    pltpu.sync_copy(dst_hbm.at[pl.ds(wid * BPT, BPT)], didx_all)
    pltpu.sync_copy(w_hbm.at[pl.ds(wid * BPT, BPT)], w_all)

    pltpu.sync_copy(y_hbm.at[pl.ds(s * NR, NR)], y_sh.at[pl.ds(s * NR, NR)])

    zero = jnp.zeros((L,), jnp.float32)

    def zero_rows(r, carry):
        for j in range(DH // L):
            rows0[r, pl.ds(j * L, L)] = zero
        return carry

    lax.fori_loop(0, K, zero_rows, 0)
    for t in range(NR // K):
        pltpu.sync_copy(rows0, acc_sh.at[pl.ds(s * NR + t * K, K)])
    plsc.subcore_barrier()

    def gather(b, p):
        pltpu.async_copy(y_sh.at[sidx_all.at[b]], rows[p], gsem[p])

    def gather_wait(b, p):
        pltpu.make_async_copy(y_sh.at[sidx_all.at[b]], rows[p], gsem[p]).wait()

    def scatter(b, p):
        pltpu.async_copy(rows[p], acc_sh.at[didx_all.at[b]], ssem[p], add=True)

    def scatter_wait(b, p):
        pltpu.make_async_copy(rows[p], acc_sh.at[didx_all.at[b]], ssem[p]).wait()

    def scale(b, p):
        def g_body(g, carry):
            wg = w_all[b, pl.ds(g * L, L)]
            for i in range(L):
                wk = wg[i]
                k = g * L + i
                for j in range(DH // L):
                    rows[p][k, pl.ds(j * L, L)] = rows[p][k, pl.ds(j * L, L)] * wk
            return carry

        lax.fori_loop(0, K // L, g_body, 0)

    gather(0, 0)

    def body(i, carry):
        for u in range(2):
            b = 2 * i + u
            p = u
            pn = 1 - u
            gather_wait(b, p)

            @pl.when(b >= 1)
            def _():
                scatter_wait(b - 1, pn)

            @pl.when(b + 1 < BPT)
            def _():
                gather(b + 1, pn)

            scale(b, p)
            scatter(b, p)
        return carry

    lax.fori_loop(0, BPT // 2, body, 0)
    scatter_wait(BPT - 1, (BPT - 1) % 2)
    plsc.subcore_barrier()
    for t in range(NR // K):
        pltpu.sync_copy(acc_sh.at[pl.ds(s * NR + t * K, K)],
                        out_hbm.at[c, pl.ds(s * NR + t * K, K)])


def _tc_prescale_body(degp_ref, x_ref, dis_ref, ya_ref, yb_ref):
    deg = degp_ref[0] + degp_ref[1]
    pos = deg > 0.0
    dis = jnp.where(pos, lax.rsqrt(jnp.where(pos, deg, 1.0)), 0.0)
    dis_ref[...] = dis
    ya_ref[...] = dis * x_ref[:, :DH]
    yb_ref[...] = dis * x_ref[:, DH:]


def _tc_mid_body(dis_ref, pa_ref, pb_ref, ya_ref, yb_ref):
    d2 = dis_ref[...] * dis_ref[...]
    ya_ref[...] = d2 * (pa_ref[0] + pa_ref[1])
    yb_ref[...] = d2 * (pb_ref[0] + pb_ref[1])


def _tc_final_body(dis_ref, qa_ref, qb_ref, h_ref):
    dis = dis_ref[...]
    h_ref[:, :DH] = dis * (qa_ref[0] + qa_ref[1])
    h_ref[:, DH:] = dis * (qb_ref[0] + qb_ref[1])


_tc_prescale = pl.pallas_call(
    _tc_prescale_body,
    out_shape=(
        jax.ShapeDtypeStruct((N_PAD, 1), jnp.float32),
        jax.ShapeDtypeStruct((N_PAD, DH), jnp.float32),
        jax.ShapeDtypeStruct((N_PAD, DH), jnp.float32),
    ),
)

_tc_mid = pl.pallas_call(
    _tc_mid_body,
    out_shape=(
        jax.ShapeDtypeStruct((N_PAD, DH), jnp.float32),
        jax.ShapeDtypeStruct((N_PAD, DH), jnp.float32),
    ),
)

_tc_final = pl.pallas_call(
    _tc_final_body,
    out_shape=jax.ShapeDtypeStruct((N_PAD, D), jnp.float32),
)


@jax.jit
def kernel(x, edge_index, edge_weight):
    src = edge_index[0].astype(jnp.int32)
    dst = edge_index[1].astype(jnp.int32)
    srcp = jnp.pad(src, (0, EP - N_EDGES)).reshape(NW * BPT, K)
    dstp = jnp.pad(dst, (0, EP - N_EDGES)).reshape(NW * BPT, K)
    wp = jnp.pad(edge_weight, (0, EP - N_EDGES)).reshape(NW * BPT, K)
    xp = jnp.pad(x, ((0, N_PAD - N_NODES), (0, 0)))

    degp = _sc_degree(dstp, wp)
    dis, ya, yb = _tc_prescale(degp[:, :, None], xp)
    pa = _sc_edge_pass(ya, srcp, dstp, wp)
    pb = _sc_edge_pass(yb, srcp, dstp, wp)
    y2a, y2b = _tc_mid(dis, pa, pb)
    qa = _sc_edge_pass(y2a, srcp, dstp, wp)
    qb = _sc_edge_pass(y2b, srcp, dstp, wp)
    h = _tc_final(dis, qa, qb)
    return h[:N_NODES]
